# Initial kernel scaffold; baseline (speedup 1.0000x reference)
import jax
import jax.numpy as jnp
from jax.experimental import pallas as pl


def kernel(target, hm, dhi, dni, ws, rh, t, r_table, dhi_table, dni_table, ws_table, rh_table, t_table, gamma, beta):
    raise NotImplementedError("write your pallas kernel here")


# speedup vs baseline: 3.5641x; 3.5641x over previous
import functools

import jax
import jax.numpy as jnp
from jax import lax
from jax.experimental import pallas as pl
from jax.experimental.pallas import tpu as pltpu
from jax.experimental.pallas import tpu_sc as plsc

B, L, D = 4096, 200, 64
N = B * L
NC, NS = 2, 16
NW = NC * NS
W = N // NW
C = 1024
KJ = C // 128
G = W // C
NT = 6


def _sc_embed_sum(idx_stacked, tables):
    mesh = plsc.VectorSubcoreMesh(core_axis_name="c", subcore_axis_name="s")

    @functools.partial(
        pl.kernel,
        out_type=jax.ShapeDtypeStruct((N, D), jnp.float32),
        mesh=mesh,
        scratch_types=[
            pltpu.VMEM((NT, KJ, 128), jnp.int32),
            pltpu.VMEM((C, D), jnp.float32),
            pltpu.SemaphoreType.DMA,
        ],
        compiler_params=pltpu.CompilerParams(use_tc_tiling_on_sc=False),
    )
    def k(idx_hbm, t0, t1, t2, t3, t4, t5, out_hbm, idx_v, acc, sem):
        tabs = [t0, t1, t2, t3, t4, t5]
        wid = lax.axis_index("s") * NC + lax.axis_index("c")

        def chunk(g, carry):
            row0 = wid * W + g * C
            j0 = pl.multiple_of(row0 // 128, 8)
            pltpu.sync_copy(idx_hbm.at[:, pl.ds(j0, KJ), :], idx_v)
            first = [
                pltpu.async_copy(
                    tabs[0].at[idx_v.at[0, j]],
                    acc.at[pl.ds(j * 128, 128)], sem)
                for j in range(KJ)
            ]
            for d in first:
                d.wait()
            rest = [
                pltpu.async_copy(
                    tabs[t].at[idx_v.at[t, j]],
                    acc.at[pl.ds(j * 128, 128)], sem, add=True)
                for t in range(1, NT) for j in range(KJ)
            ]
            for d in rest:
                d.wait()
            pltpu.sync_copy(acc, out_hbm.at[pl.ds(row0, C)])
            return carry

        lax.fori_loop(0, G, chunk, 0)

    return k(idx_stacked, *tables)


R = 2048


def _tc_layernorm(hm2, esum, gamma, beta):
    def body(hm_ref, e_ref, g_ref, b_ref, o_ref):
        x = hm_ref[...] + e_ref[...]
        mu = jnp.mean(x, axis=1, keepdims=True)
        xc = x - mu
        var = jnp.mean(xc * xc, axis=1, keepdims=True)
        inv = lax.rsqrt(var + 1e-5)
        o_ref[...] = xc * inv * g_ref[...] + b_ref[...]

    return pl.pallas_call(
        body,
        grid=(N // R,),
        in_specs=[
            pl.BlockSpec((R, D), lambda i: (i, 0)),
            pl.BlockSpec((R, D), lambda i: (i, 0)),
            pl.BlockSpec((1, D), lambda i: (0, 0)),
            pl.BlockSpec((1, D), lambda i: (0, 0)),
        ],
        out_specs=pl.BlockSpec((R, D), lambda i: (i, 0)),
        out_shape=jax.ShapeDtypeStruct((N, D), jnp.float32),
    )(hm2, esum, gamma, beta)


def kernel(target, hm, dhi, dni, ws, rh, t, r_table, dhi_table, dni_table,
           ws_table, rh_table, t_table, gamma, beta):
    idx = jnp.stack([
        target.reshape(N), dhi.reshape(N), dni.reshape(N),
        ws.reshape(N), rh.reshape(N), t.reshape(N),
    ]).astype(jnp.int32).reshape(NT, N // 128, 128)
    tables = [r_table, dhi_table, dni_table, ws_table, rh_table, t_table]
    esum = _sc_embed_sum(idx, tables)
    out = _tc_layernorm(
        hm.reshape(N, D), esum,
        gamma.reshape(1, D), beta.reshape(1, D))
    return out.reshape(B, L, D)
